# Initial kernel scaffold; baseline (speedup 1.0000x reference)
import jax
import jax.numpy as jnp
from jax.experimental import pallas as pl


def kernel(x, Wsp, bsp, Wfc, bfc):
    raise NotImplementedError("write your pallas kernel here")


# speedup vs baseline: 15.8502x; 15.8502x over previous
import jax
import jax.numpy as jnp
from jax.experimental import pallas as pl

_IN = 784
_NC = 10
_K = 16
_NEG = -1e30
_TB = 1024


def _fused_kernel(x_ref, wsp_ref, bsp_ref, wfct_ref, bfc_ref, out_ref):
    wfct = wfct_ref[...]
    rows = jax.lax.broadcasted_iota(jnp.int32, (_IN, _K), 0)
    m782 = rows == 782
    g0 = jnp.where(m782, jnp.broadcast_to(wfct[0:1, :], (_IN, _K)), wfct)
    rolled = jnp.concatenate([wfct[1:_IN, :], wfct[0:1, :]], axis=0)
    g1 = jnp.where(m782, jnp.broadcast_to(wfct[1:2, :], (_IN, _K)), rolled)

    wsp = wsp_ref[...]
    bsp = bsp_ref[...]
    weff = wsp[:, 0:1] * g0 + wsp[:, 1:2] * g1
    beff = jnp.sum(bsp[:, 0:1] * g0 + bsp[:, 1:2] * g1,
                   axis=0, keepdims=True) + bfc_ref[...]

    logits = jax.lax.dot_general(
        x_ref[...], weff, (((1,), (0,)), ((), ())),
        preferred_element_type=jnp.float32) + beff
    m = jnp.max(logits, axis=1, keepdims=True)
    e = jnp.exp(logits - m)
    out_ref[...] = e / jnp.sum(e, axis=1, keepdims=True)


def _run(x, Wsp, bsp, wfct, bfc_p, interpret=False):
    B = x.shape[0]
    return pl.pallas_call(
        _fused_kernel,
        grid=(B // _TB,),
        in_specs=[
            pl.BlockSpec((_TB, _IN), lambda i: (i, 0)),
            pl.BlockSpec((_IN, 2), lambda i: (0, 0)),
            pl.BlockSpec((_IN, 2), lambda i: (0, 0)),
            pl.BlockSpec((_IN, _K), lambda i: (0, 0)),
            pl.BlockSpec((1, _K), lambda i: (0, 0)),
        ],
        out_specs=pl.BlockSpec((_TB, _K), lambda i: (i, 0)),
        out_shape=jax.ShapeDtypeStruct((B, _K), jnp.float32),
        interpret=interpret,
    )(x, Wsp, bsp, wfct, bfc_p)


@jax.jit
def _kernel_jit(x, Wsp, bsp, Wfc, bfc):
    x = x.reshape(x.shape[0], -1)
    wfct = jnp.zeros((_IN, _K), jnp.float32).at[:, :_NC].set(Wfc.T)
    bfc_p = jnp.full((1, _K), _NEG, jnp.float32).at[0, :_NC].set(bfc)
    out = _run(x, Wsp, bsp, wfct, bfc_p)
    return out[:, :_NC]


def kernel(x, Wsp, bsp, Wfc, bfc):
    return _kernel_jit(x, Wsp, bsp, Wfc, bfc)


# speedup vs baseline: 16.7724x; 1.0582x over previous
import jax
import jax.numpy as jnp
from jax.experimental import pallas as pl
from jax.experimental.pallas import tpu as pltpu

_IN = 784
_NC = 10
_K = 16
_NEG = -1e30
_TB = 2048


def _fused_kernel(x_ref, wsp_ref, bsp_ref, wfct_ref, bfc_ref, out_ref,
                  weff_ref, beff_ref):
    @pl.when(pl.program_id(0) == 0)
    def _build_weights():
        wfct = wfct_ref[...]
        rows = jax.lax.broadcasted_iota(jnp.int32, (_IN, _K), 0)
        m782 = rows == 782
        g0 = jnp.where(m782, jnp.broadcast_to(wfct[0:1, :], (_IN, _K)), wfct)
        rolled = jnp.concatenate([wfct[1:_IN, :], wfct[0:1, :]], axis=0)
        g1 = jnp.where(m782, jnp.broadcast_to(wfct[1:2, :], (_IN, _K)), rolled)

        wsp = wsp_ref[...]
        bsp = bsp_ref[...]
        weff_ref[...] = wsp[:, 0:1] * g0 + wsp[:, 1:2] * g1
        beff_ref[...] = jnp.sum(bsp[:, 0:1] * g0 + bsp[:, 1:2] * g1,
                                axis=0, keepdims=True) + bfc_ref[...]

    logits = jax.lax.dot_general(
        x_ref[...], weff_ref[...], (((1,), (0,)), ((), ())),
        preferred_element_type=jnp.float32) + beff_ref[...]
    m = jnp.max(logits, axis=1, keepdims=True)
    e = jnp.exp(logits - m)
    out_ref[...] = e / jnp.sum(e, axis=1, keepdims=True)


def _run(x, Wsp, bsp, wfct, bfc_p, interpret=False):
    B = x.shape[0]
    return pl.pallas_call(
        _fused_kernel,
        grid=(B // _TB,),
        in_specs=[
            pl.BlockSpec((_TB, _IN), lambda i: (i, 0)),
            pl.BlockSpec((_IN, 2), lambda i: (0, 0)),
            pl.BlockSpec((_IN, 2), lambda i: (0, 0)),
            pl.BlockSpec((_IN, _K), lambda i: (0, 0)),
            pl.BlockSpec((1, _K), lambda i: (0, 0)),
        ],
        out_specs=pl.BlockSpec((_TB, _K), lambda i: (i, 0)),
        out_shape=jax.ShapeDtypeStruct((B, _K), jnp.float32),
        scratch_shapes=[
            pltpu.VMEM((_IN, _K), jnp.float32),
            pltpu.VMEM((1, _K), jnp.float32),
        ],
        compiler_params=pltpu.CompilerParams(
            dimension_semantics=("arbitrary",)),
        interpret=interpret,
    )(x, Wsp, bsp, wfct, bfc_p)


@jax.jit
def _kernel_jit(x, Wsp, bsp, Wfc, bfc):
    x = x.reshape(x.shape[0], -1)
    wfct = jnp.zeros((_IN, _K), jnp.float32).at[:, :_NC].set(Wfc.T)
    bfc_p = jnp.full((1, _K), _NEG, jnp.float32).at[0, :_NC].set(bfc)
    out = _run(x, Wsp, bsp, wfct, bfc_p)
    return out[:, :_NC]


def kernel(x, Wsp, bsp, Wfc, bfc):
    return _kernel_jit(x, Wsp, bsp, Wfc, bfc)
